# Initial kernel scaffold; baseline (speedup 1.0000x reference)
import jax
import jax.numpy as jnp
from jax.experimental import pallas as pl


def kernel(x, edge_index, edge_attr, W1, root1, b1, W2, root2, b2):
    raise NotImplementedError("write your pallas kernel here")


# speedup vs baseline: 2.2305x; 2.2305x over previous
import jax
import jax.numpy as jnp
from jax import lax
from jax.experimental import pallas as pl
from jax.experimental.pallas import tpu as pltpu
from jax.experimental.pallas import tpu_sc as plsc

_N = 10000
_E = 320000
_IN = 128
_HID = 256
_OUT = 128
_NC = 2
_NS = 16
_L = 16
_B = 80
_NP = 10240
_STRIPE = _NP // _NS

_mesh = plsc.VectorSubcoreMesh(core_axis_name="c", subcore_axis_name="s")


def _sc_layer1_body(x_hbm, src_hbm, dst_hbm, frac_hbm, zrow_hbm, zdeg_hbm,
                    s_hbm, deg_hbm,
                    idx_s, idx_d, frb_v, rows_v, ones_v, acc, dacc, sem):
  c = lax.axis_index("c")
  s = lax.axis_index("s")
  r0 = s * _STRIPE

  pltpu.sync_copy(zrow_hbm.at[pl.ds(r0, _STRIPE)], acc.at[pl.ds(r0, _STRIPE)])

  @pl.when(c == 0)
  def _():
    pltpu.sync_copy(zdeg_hbm.at[pl.ds(r0, _STRIPE)],
                    dacc.at[pl.ds(r0, _STRIPE)])

  @pl.loop(0, _B // _L)
  def _(g):
    ones_v[pl.ds(g * _L, _L)] = jnp.full((_L,), 1.0, jnp.float32)

  plsc.subcore_barrier()

  per_tile = _E // _NS
  base = s * per_tile

  @pl.loop(0, per_tile // _B)
  def _(j):
    off = base + j * _B
    pltpu.sync_copy(src_hbm.at[pl.ds(off, _B)], idx_s.at[0])
    pltpu.sync_copy(dst_hbm.at[pl.ds(off, _B)], idx_d.at[0])
    pltpu.async_copy(x_hbm.at[idx_s.at[0]], rows_v, sem).wait()

    @pl.when(c == 1)
    def _():
      pltpu.sync_copy(frac_hbm.at[pl.ds(off, _B)], frb_v)

      @pl.loop(0, _B)
      def _(e):
        fs = frb_v[e]
        for cc in range(_IN // _L):
          rows_v[e, pl.ds(cc * _L, _L)] = rows_v[e, pl.ds(cc * _L, _L)] * fs

    pltpu.sync_copy(rows_v, acc.at[idx_d.at[0]], add=True)

    @pl.when(c == 0)
    def _():
      pltpu.sync_copy(ones_v, dacc.at[idx_d.at[0]], add=True)

  plsc.subcore_barrier()
  pltpu.sync_copy(acc.at[pl.ds(r0, _STRIPE)], s_hbm.at[c, pl.ds(r0, _STRIPE)])

  @pl.when(c == 0)
  def _():
    pltpu.sync_copy(dacc.at[pl.ds(r0, _STRIPE)],
                    deg_hbm.at[pl.ds(r0, _STRIPE)])


def _sc_layer2_body(y_hbm, src_hbm, dst_hbm, frac_hbm, zrow_hbm,
                    p_hbm,
                    idx_s, idx_d, frb_v, grows_v, crows_v, acc, sem):
  c = lax.axis_index("c")
  s = lax.axis_index("s")
  r0 = s * _STRIPE
  pltpu.sync_copy(zrow_hbm.at[pl.ds(r0, _STRIPE)], acc.at[pl.ds(r0, _STRIPE)])
  plsc.subcore_barrier()

  wid = s * _NC + c
  per_w = _E // (_NC * _NS)
  base = wid * per_w

  @pl.loop(0, per_w // _B)
  def _(j):
    off = base + j * _B
    pltpu.sync_copy(src_hbm.at[pl.ds(off, _B)], idx_s.at[0])
    pltpu.sync_copy(dst_hbm.at[pl.ds(off, _B)], idx_d.at[0])
    pltpu.sync_copy(frac_hbm.at[pl.ds(off, _B)], frb_v)
    pltpu.async_copy(y_hbm.at[idx_s.at[0]], grows_v, sem).wait()

    @pl.loop(0, _B)
    def _(e):
      fs = frb_v[e]
      for cc in range(_OUT // _L):
        a = grows_v[e, pl.ds(cc * _L, _L)]
        b = grows_v[e, pl.ds(_OUT + cc * _L, _L)]
        crows_v[e, pl.ds(cc * _L, _L)] = a + fs * b

    pltpu.sync_copy(crows_v, acc.at[idx_d.at[0]], add=True)

  plsc.subcore_barrier()
  pltpu.sync_copy(acc.at[pl.ds(r0, _STRIPE)], p_hbm.at[c, pl.ds(r0, _STRIPE)])


_BN = 1000


def _tc1_body(s0_ref, s1_ref, deg_ref, x_ref, w0_ref, dw_ref, r1_ref, b1_ref,
              w2c_ref, h_ref, y_ref):
  recip = 1.0 / jnp.maximum(deg_ref[...][:, 0:1], 1.0)
  agg = (jnp.dot(s0_ref[...], w0_ref[...], preferred_element_type=jnp.float32)
         + jnp.dot(s1_ref[...], dw_ref[...],
                   preferred_element_type=jnp.float32))
  t = (agg * recip
       + jnp.dot(x_ref[...], r1_ref[...], preferred_element_type=jnp.float32)
       + b1_ref[...])
  h = jnp.where(t > 0, t, jnp.exp(t) - 1.0)
  h_ref[...] = h
  y_ref[...] = jnp.dot(h, w2c_ref[...], preferred_element_type=jnp.float32)


def _tc2_body(p0_ref, p1_ref, deg_ref, h_ref, r2_ref, b2_ref, o_ref):
  recip = 1.0 / jnp.maximum(deg_ref[...][:, 0:1], 1.0)
  o = ((p0_ref[...] + p1_ref[...]) * recip
       + jnp.dot(h_ref[...], r2_ref[...], preferred_element_type=jnp.float32)
       + b2_ref[...])
  m = jnp.max(o, axis=1, keepdims=True)
  o_ref[...] = (o - m) - jnp.log(
      jnp.sum(jnp.exp(o - m), axis=1, keepdims=True))


def kernel(x, edge_index, edge_attr, W1, root1, b1, W2, root2, b2):
  src = edge_index[0]
  dst = edge_index[1]
  frac = jnp.broadcast_to(edge_attr[:, 0:1], (_E, _L))
  zrow = jnp.zeros((_NP, _IN), jnp.float32)
  zdeg = jnp.zeros((_NP,), jnp.float32)

  sc1 = pl.kernel(
      _sc_layer1_body,
      out_type=[jax.ShapeDtypeStruct((_NC, _NP, _IN), jnp.float32),
                jax.ShapeDtypeStruct((_NP,), jnp.float32)],
      mesh=_mesh,
      scratch_types=[
          pltpu.VMEM((1, _B), jnp.int32),
          pltpu.VMEM((1, _B), jnp.int32),
          pltpu.VMEM((_B, _L), jnp.float32),
          pltpu.VMEM((_B, _IN), jnp.float32),
          pltpu.VMEM((_B,), jnp.float32),
          pltpu.VMEM_SHARED((_NP, _IN), jnp.float32),
          pltpu.VMEM_SHARED((_NP,), jnp.float32),
          pltpu.SemaphoreType.DMA,
      ],
  )
  S, deg1 = sc1(x, src, dst, frac, zrow, zdeg)
  S0 = S[0, :_N, :]
  S1 = S[1, :_N, :]
  degc = deg1[:_N].reshape(_N, 1)

  grid = (_N // _BN,)
  h, Y = pl.pallas_call(
      _tc1_body,
      grid=grid,
      in_specs=[
          pl.BlockSpec((_BN, _IN), lambda i: (i, 0)),
          pl.BlockSpec((_BN, _IN), lambda i: (i, 0)),
          pl.BlockSpec((_BN, 1), lambda i: (i, 0)),
          pl.BlockSpec((_BN, _IN), lambda i: (i, 0)),
          pl.BlockSpec((_IN, _HID), lambda i: (0, 0)),
          pl.BlockSpec((_IN, _HID), lambda i: (0, 0)),
          pl.BlockSpec((_IN, _HID), lambda i: (0, 0)),
          pl.BlockSpec((1, _HID), lambda i: (0, 0)),
          pl.BlockSpec((_HID, 2 * _OUT), lambda i: (0, 0)),
      ],
      out_specs=[
          pl.BlockSpec((_BN, _HID), lambda i: (i, 0)),
          pl.BlockSpec((_BN, 2 * _OUT), lambda i: (i, 0)),
      ],
      out_shape=[
          jax.ShapeDtypeStruct((_N, _HID), jnp.float32),
          jax.ShapeDtypeStruct((_N, 2 * _OUT), jnp.float32),
      ],
  )(S0, S1, degc, x, W1[0], W1[1] - W1[0], root1,
    b1.reshape(1, _HID), jnp.concatenate([W2[0], W2[1] - W2[0]], axis=1))

  sc2 = pl.kernel(
      _sc_layer2_body,
      out_type=jax.ShapeDtypeStruct((_NC, _NP, _OUT), jnp.float32),
      mesh=_mesh,
      scratch_types=[
          pltpu.VMEM((1, _B), jnp.int32),
          pltpu.VMEM((1, _B), jnp.int32),
          pltpu.VMEM((_B, _L), jnp.float32),
          pltpu.VMEM((_B, 2 * _OUT), jnp.float32),
          pltpu.VMEM((_B, _OUT), jnp.float32),
          pltpu.VMEM_SHARED((_NP, _OUT), jnp.float32),
          pltpu.SemaphoreType.DMA,
      ],
  )
  P = sc2(Y, src, dst, frac, zrow)

  out = pl.pallas_call(
      _tc2_body,
      grid=grid,
      in_specs=[
          pl.BlockSpec((_BN, _OUT), lambda i: (i, 0)),
          pl.BlockSpec((_BN, _OUT), lambda i: (i, 0)),
          pl.BlockSpec((_BN, 1), lambda i: (i, 0)),
          pl.BlockSpec((_BN, _HID), lambda i: (i, 0)),
          pl.BlockSpec((_HID, _OUT), lambda i: (0, 0)),
          pl.BlockSpec((1, _OUT), lambda i: (0, 0)),
      ],
      out_specs=pl.BlockSpec((_BN, _OUT), lambda i: (i, 0)),
      out_shape=jax.ShapeDtypeStruct((_N, _OUT), jnp.float32),
  )(P[0, :_N, :], P[1, :_N, :], degc, h, root2, b2.reshape(1, _OUT))

  return out


# speedup vs baseline: 4.2856x; 1.9214x over previous
import jax
import jax.numpy as jnp
from jax import lax
from jax.experimental import pallas as pl
from jax.experimental.pallas import tpu as pltpu
from jax.experimental.pallas import tpu_sc as plsc

_N = 10000
_E = 320000
_IN = 128
_HID = 256
_OUT = 128
_NC = 2
_NS = 16
_L = 16
_B1 = 80
_B2 = 40
_NP = 10240
_STRIPE = _NP // _NS

_mesh = plsc.VectorSubcoreMesh(core_axis_name="c", subcore_axis_name="s")


def _sc_layer1_body(x_hbm, src_hbm, dst_hbm, frac_hbm, zrow_hbm, zdeg_hbm,
                    s_hbm, deg_hbm,
                    idx_s, idx_d, frb_v, rows_v, ones_v, acc, dacc,
                    sg0, sg1, ss0, ss1, sdf0, sdf1):
  c = lax.axis_index("c")
  s = lax.axis_index("s")
  r0 = s * _STRIPE
  per_tile = _E // _NS
  base = s * per_tile
  nch = per_tile // _B1
  sg = (sg0, sg1)
  ss = (ss0, ss1)
  sdf = (sdf0, sdf1)

  pltpu.sync_copy(zrow_hbm.at[pl.ds(r0, _STRIPE)], acc.at[pl.ds(r0, _STRIPE)])

  @pl.when(c == 0)
  def _():
    pltpu.sync_copy(zdeg_hbm.at[pl.ds(r0, _STRIPE)],
                    dacc.at[pl.ds(r0, _STRIPE)])

  @pl.loop(0, _B1 // _L)
  def _(g):
    ones_v[pl.ds(g * _L, _L)] = jnp.full((_L,), 1.0, jnp.float32)

  plsc.subcore_barrier()

  def issue_src(j, b):
    pltpu.async_copy(src_hbm.at[pl.ds(base + j * _B1, _B1)],
                     idx_s.at[b], ss[b])

  def issue_df(j, b):
    pltpu.async_copy(dst_hbm.at[pl.ds(base + j * _B1, _B1)],
                     idx_d.at[b], sdf[b])

    @pl.when(c == 1)
    def _():
      pltpu.async_copy(frac_hbm.at[pl.ds(base + j * _B1, _B1)],
                       frb_v.at[b], sdf[b])

  def issue_gather(j, b):
    pltpu.async_copy(x_hbm.at[idx_s.at[b]], rows_v.at[b], sg[b])

  def step(j, b, nb):
    pltpu.make_async_copy(x_hbm.at[idx_s.at[b]], rows_v.at[b], sg[b]).wait()

    @pl.when(j + 1 < nch)
    def _():
      pltpu.make_async_copy(src_hbm.at[pl.ds(0, _B1)], idx_s.at[nb],
                            ss[nb]).wait()
      issue_gather(j + 1, nb)

    @pl.when(j + 2 < nch)
    def _():
      issue_src(j + 2, b)

    pltpu.make_async_copy(dst_hbm.at[pl.ds(0, _B1)], idx_d.at[b],
                          sdf[b]).wait()

    @pl.when(c == 1)
    def _():
      pltpu.make_async_copy(frac_hbm.at[pl.ds(0, _B1)], frb_v.at[b],
                            sdf[b]).wait()

      @pl.loop(0, _B1)
      def _(e):
        fs = frb_v[b, e]
        for cc in range(_IN // _L):
          rows_v[b, e, pl.ds(cc * _L, _L)] = (
              rows_v[b, e, pl.ds(cc * _L, _L)] * fs)

    pltpu.sync_copy(rows_v.at[b], acc.at[idx_d.at[b]], add=True)

    @pl.when(c == 0)
    def _():
      pltpu.sync_copy(ones_v, dacc.at[idx_d.at[b]], add=True)

    @pl.when(j + 2 < nch)
    def _():
      issue_df(j + 2, b)

  issue_src(0, 0)
  issue_src(1, 1)
  issue_df(0, 0)
  issue_df(1, 1)
  pltpu.make_async_copy(src_hbm.at[pl.ds(0, _B1)], idx_s.at[0], ss0).wait()
  issue_gather(0, 0)

  @pl.loop(0, nch // 2)
  def _(jj):
    j0 = 2 * jj
    step(j0, 0, 1)
    step(j0 + 1, 1, 0)

  plsc.subcore_barrier()
  pltpu.sync_copy(acc.at[pl.ds(r0, _STRIPE)], s_hbm.at[c, pl.ds(r0, _STRIPE)])

  @pl.when(c == 0)
  def _():
    pltpu.sync_copy(dacc.at[pl.ds(r0, _STRIPE)],
                    deg_hbm.at[pl.ds(r0, _STRIPE)])


def _sc_layer2_body(y_hbm, src_hbm, dst_hbm, frac_hbm, zrow_hbm,
                    p_hbm,
                    idx_s, idx_d, frb_v, grows_v, crows_v, acc,
                    sg0, sg1, ss0, ss1, sdf0, sdf1):
  c = lax.axis_index("c")
  s = lax.axis_index("s")
  r0 = s * _STRIPE
  per_w = _E // (_NC * _NS)
  base = (s * _NC + c) * per_w
  nch = per_w // _B2
  sg = (sg0, sg1)
  ss = (ss0, ss1)
  sdf = (sdf0, sdf1)

  pltpu.sync_copy(zrow_hbm.at[pl.ds(r0, _STRIPE)], acc.at[pl.ds(r0, _STRIPE)])
  plsc.subcore_barrier()

  def issue_src(j, b):
    pltpu.async_copy(src_hbm.at[pl.ds(base + j * _B2, _B2)],
                     idx_s.at[b], ss[b])

  def issue_df(j, b):
    pltpu.async_copy(dst_hbm.at[pl.ds(base + j * _B2, _B2)],
                     idx_d.at[b], sdf[b])
    pltpu.async_copy(frac_hbm.at[pl.ds(base + j * _B2, _B2)],
                     frb_v.at[b], sdf[b])

  def issue_gather(j, b):
    pltpu.async_copy(y_hbm.at[idx_s.at[b]], grows_v.at[b], sg[b])

  def step(j, b, nb):
    pltpu.make_async_copy(y_hbm.at[idx_s.at[b]], grows_v.at[b], sg[b]).wait()

    @pl.when(j + 1 < nch)
    def _():
      pltpu.make_async_copy(src_hbm.at[pl.ds(0, _B2)], idx_s.at[nb],
                            ss[nb]).wait()
      issue_gather(j + 1, nb)

    @pl.when(j + 2 < nch)
    def _():
      issue_src(j + 2, b)

    pltpu.make_async_copy(dst_hbm.at[pl.ds(0, _B2)], idx_d.at[b],
                          sdf[b]).wait()
    pltpu.make_async_copy(frac_hbm.at[pl.ds(0, _B2)], frb_v.at[b],
                          sdf[b]).wait()

    @pl.loop(0, _B2)
    def _(e):
      fs = frb_v[b, e]
      for cc in range(_OUT // _L):
        a = grows_v[b, e, pl.ds(cc * _L, _L)]
        bb = grows_v[b, e, pl.ds(_OUT + cc * _L, _L)]
        crows_v[e, pl.ds(cc * _L, _L)] = a + fs * bb

    pltpu.sync_copy(crows_v, acc.at[idx_d.at[b]], add=True)

    @pl.when(j + 2 < nch)
    def _():
      issue_df(j + 2, b)

  issue_src(0, 0)
  issue_src(1, 1)
  issue_df(0, 0)
  issue_df(1, 1)
  pltpu.make_async_copy(src_hbm.at[pl.ds(0, _B2)], idx_s.at[0], ss0).wait()
  issue_gather(0, 0)

  @pl.loop(0, nch // 2)
  def _(jj):
    j0 = 2 * jj
    step(j0, 0, 1)
    step(j0 + 1, 1, 0)

  plsc.subcore_barrier()
  pltpu.sync_copy(acc.at[pl.ds(r0, _STRIPE)], p_hbm.at[c, pl.ds(r0, _STRIPE)])


_BN = 1000


def _tc1_body(s0_ref, s1_ref, deg_ref, x_ref, w0_ref, dw_ref, r1_ref, b1_ref,
              w2c_ref, h_ref, y_ref):
  recip = 1.0 / jnp.maximum(deg_ref[...], 1.0)
  agg = (jnp.dot(s0_ref[...], w0_ref[...], preferred_element_type=jnp.float32)
         + jnp.dot(s1_ref[...], dw_ref[...],
                   preferred_element_type=jnp.float32))
  t = (agg * recip
       + jnp.dot(x_ref[...], r1_ref[...], preferred_element_type=jnp.float32)
       + b1_ref[...])
  h = jnp.where(t > 0, t, jnp.exp(t) - 1.0)
  h_ref[...] = h
  y_ref[...] = jnp.dot(h, w2c_ref[...], preferred_element_type=jnp.float32)


def _tc2_body(p0_ref, p1_ref, deg_ref, h_ref, r2_ref, b2_ref, o_ref):
  recip = 1.0 / jnp.maximum(deg_ref[...], 1.0)
  o = ((p0_ref[...] + p1_ref[...]) * recip
       + jnp.dot(h_ref[...], r2_ref[...], preferred_element_type=jnp.float32)
       + b2_ref[...])
  m = jnp.max(o, axis=1, keepdims=True)
  o_ref[...] = (o - m) - jnp.log(
      jnp.sum(jnp.exp(o - m), axis=1, keepdims=True))


def kernel(x, edge_index, edge_attr, W1, root1, b1, W2, root2, b2):
  src = edge_index[0]
  dst = edge_index[1]
  frac = jnp.broadcast_to(edge_attr[:, 0:1], (_E, _L))
  zrow = jnp.zeros((_NP, _IN), jnp.float32)
  zdeg = jnp.zeros((_NP,), jnp.float32)

  sc1 = pl.kernel(
      _sc_layer1_body,
      out_type=[jax.ShapeDtypeStruct((_NC, _NP, _IN), jnp.float32),
                jax.ShapeDtypeStruct((_NP,), jnp.float32)],
      mesh=_mesh,
      scratch_types=[
          pltpu.VMEM((2, _B1), jnp.int32),
          pltpu.VMEM((2, _B1), jnp.int32),
          pltpu.VMEM((2, _B1, _L), jnp.float32),
          pltpu.VMEM((2, _B1, _IN), jnp.float32),
          pltpu.VMEM((_B1,), jnp.float32),
          pltpu.VMEM_SHARED((_NP, _IN), jnp.float32),
          pltpu.VMEM_SHARED((_NP,), jnp.float32),
          pltpu.SemaphoreType.DMA,
          pltpu.SemaphoreType.DMA,
          pltpu.SemaphoreType.DMA,
          pltpu.SemaphoreType.DMA,
          pltpu.SemaphoreType.DMA,
          pltpu.SemaphoreType.DMA,
      ],
  )
  S, deg1 = sc1(x, src, dst, frac, zrow, zdeg)
  S0 = S[0, :_N, :]
  S1 = S[1, :_N, :]
  degc = deg1[:_N].reshape(_N, 1)

  grid = (_N // _BN,)
  h, Y = pl.pallas_call(
      _tc1_body,
      grid=grid,
      in_specs=[
          pl.BlockSpec((_BN, _IN), lambda i: (i, 0)),
          pl.BlockSpec((_BN, _IN), lambda i: (i, 0)),
          pl.BlockSpec((_BN, 1), lambda i: (i, 0)),
          pl.BlockSpec((_BN, _IN), lambda i: (i, 0)),
          pl.BlockSpec((_IN, _HID), lambda i: (0, 0)),
          pl.BlockSpec((_IN, _HID), lambda i: (0, 0)),
          pl.BlockSpec((_IN, _HID), lambda i: (0, 0)),
          pl.BlockSpec((1, _HID), lambda i: (0, 0)),
          pl.BlockSpec((_HID, 2 * _OUT), lambda i: (0, 0)),
      ],
      out_specs=[
          pl.BlockSpec((_BN, _HID), lambda i: (i, 0)),
          pl.BlockSpec((_BN, 2 * _OUT), lambda i: (i, 0)),
      ],
      out_shape=[
          jax.ShapeDtypeStruct((_N, _HID), jnp.float32),
          jax.ShapeDtypeStruct((_N, 2 * _OUT), jnp.float32),
      ],
  )(S0, S1, degc, x, W1[0], W1[1] - W1[0], root1,
    b1.reshape(1, _HID), jnp.concatenate([W2[0], W2[1] - W2[0]], axis=1))

  sc2 = pl.kernel(
      _sc_layer2_body,
      out_type=jax.ShapeDtypeStruct((_NC, _NP, _OUT), jnp.float32),
      mesh=_mesh,
      scratch_types=[
          pltpu.VMEM((2, _B2), jnp.int32),
          pltpu.VMEM((2, _B2), jnp.int32),
          pltpu.VMEM((2, _B2, _L), jnp.float32),
          pltpu.VMEM((2, _B2, 2 * _OUT), jnp.float32),
          pltpu.VMEM((_B2, _OUT), jnp.float32),
          pltpu.VMEM_SHARED((_NP, _OUT), jnp.float32),
          pltpu.SemaphoreType.DMA,
          pltpu.SemaphoreType.DMA,
          pltpu.SemaphoreType.DMA,
          pltpu.SemaphoreType.DMA,
          pltpu.SemaphoreType.DMA,
          pltpu.SemaphoreType.DMA,
      ],
  )
  P = sc2(Y, src, dst, frac, zrow)

  out = pl.pallas_call(
      _tc2_body,
      grid=grid,
      in_specs=[
          pl.BlockSpec((_BN, _OUT), lambda i: (i, 0)),
          pl.BlockSpec((_BN, _OUT), lambda i: (i, 0)),
          pl.BlockSpec((_BN, 1), lambda i: (i, 0)),
          pl.BlockSpec((_BN, _HID), lambda i: (i, 0)),
          pl.BlockSpec((_HID, _OUT), lambda i: (0, 0)),
          pl.BlockSpec((1, _OUT), lambda i: (0, 0)),
      ],
      out_specs=pl.BlockSpec((_BN, _OUT), lambda i: (i, 0)),
      out_shape=jax.ShapeDtypeStruct((_N, _OUT), jnp.float32),
  )(P[0, :_N, :], P[1, :_N, :], degc, h, root2, b2.reshape(1, _OUT))

  return out


# speedup vs baseline: 4.5860x; 1.0701x over previous
import jax
import jax.numpy as jnp
from jax import lax
from jax.experimental import pallas as pl
from jax.experimental.pallas import tpu as pltpu
from jax.experimental.pallas import tpu_sc as plsc

_N = 10000
_E = 320000
_IN = 128
_HID = 256
_OUT = 128
_NC = 2
_NS = 16
_L = 16
_B1 = 80
_B2 = 40
_NP = 10240
_STRIPE = _NP // _NS

_mesh = plsc.VectorSubcoreMesh(core_axis_name="c", subcore_axis_name="s")


def _sc_layer1_body(x_hbm, src_hbm, dst_hbm, frac_hbm, zrow_hbm, zdeg_hbm,
                    s_hbm, deg_hbm,
                    idx_s, idx_d, frb_v, rows_v, ones_v, acc, dacc,
                    sg0, sg1, ss0, ss1, sdf0, sdf1):
  c = lax.axis_index("c")
  s = lax.axis_index("s")
  r0 = s * _STRIPE
  per_tile = _E // _NS
  base = s * per_tile
  nch = per_tile // _B1
  sg = (sg0, sg1)
  ss = (ss0, ss1)
  sdf = (sdf0, sdf1)

  pltpu.sync_copy(zrow_hbm.at[pl.ds(r0, _STRIPE)], acc.at[pl.ds(r0, _STRIPE)])

  @pl.when(c == 0)
  def _():
    pltpu.sync_copy(zdeg_hbm.at[pl.ds(r0, _STRIPE)],
                    dacc.at[pl.ds(r0, _STRIPE)])

  @pl.loop(0, _B1 // _L)
  def _(g):
    ones_v[pl.ds(g * _L, _L)] = jnp.full((_L,), 1.0, jnp.float32)

  plsc.subcore_barrier()

  def issue_src(j, b):
    pltpu.async_copy(src_hbm.at[pl.ds(base + j * _B1, _B1)],
                     idx_s.at[b], ss[b])

  def issue_df(j, b):
    pltpu.async_copy(dst_hbm.at[pl.ds(base + j * _B1, _B1)],
                     idx_d.at[b], sdf[b])

    @pl.when(c == 1)
    def _():
      pltpu.async_copy(frac_hbm.at[pl.ds(base + j * _B1, _B1)],
                       frb_v.at[b], sdf[b])

  def issue_gather(j, b):
    pltpu.async_copy(x_hbm.at[idx_s.at[b]], rows_v.at[b], sg[b])

  def step(j, b, nb):
    pltpu.make_async_copy(x_hbm.at[idx_s.at[b]], rows_v.at[b], sg[b]).wait()

    @pl.when(j + 1 < nch)
    def _():
      pltpu.make_async_copy(src_hbm.at[pl.ds(0, _B1)], idx_s.at[nb],
                            ss[nb]).wait()
      issue_gather(j + 1, nb)

    @pl.when(j + 2 < nch)
    def _():
      issue_src(j + 2, b)

    pltpu.make_async_copy(dst_hbm.at[pl.ds(0, _B1)], idx_d.at[b],
                          sdf[b]).wait()

    @pl.when(c == 1)
    def _():
      pltpu.make_async_copy(frac_hbm.at[pl.ds(0, _B1)], frb_v.at[b],
                            sdf[b]).wait()

      @pl.loop(0, _B1)
      def _(e):
        fs = frb_v[b, e]
        for cc in range(_IN // _L):
          rows_v[b, e, pl.ds(cc * _L, _L)] = (
              rows_v[b, e, pl.ds(cc * _L, _L)] * fs)

    pltpu.sync_copy(rows_v.at[b], acc.at[idx_d.at[b]], add=True)

    @pl.when(c == 0)
    def _():
      pltpu.sync_copy(ones_v, dacc.at[idx_d.at[b]], add=True)

    @pl.when(j + 2 < nch)
    def _():
      issue_df(j + 2, b)

  issue_src(0, 0)
  issue_src(1, 1)
  issue_df(0, 0)
  issue_df(1, 1)
  pltpu.make_async_copy(src_hbm.at[pl.ds(0, _B1)], idx_s.at[0], ss0).wait()
  issue_gather(0, 0)

  @pl.loop(0, nch // 2)
  def _(jj):
    j0 = 2 * jj
    step(j0, 0, 1)
    step(j0 + 1, 1, 0)

  plsc.subcore_barrier()
  pltpu.sync_copy(acc.at[pl.ds(r0, _STRIPE)], s_hbm.at[c, pl.ds(r0, _STRIPE)])

  @pl.when(c == 0)
  def _():
    pltpu.sync_copy(dacc.at[pl.ds(r0, _STRIPE)],
                    deg_hbm.at[pl.ds(r0, _STRIPE)])


def _sc_layer2_body(y_hbm, src_hbm, dst_hbm, frac_hbm, zrow_hbm,
                    p_hbm,
                    idx_s, idx_d, frb_v, grows_v, crows_v, acc,
                    sg0, sg1, ss0, ss1, sdf0, sdf1, ssc0, ssc1):
  c = lax.axis_index("c")
  s = lax.axis_index("s")
  r0 = s * _STRIPE
  per_w = _E // (_NC * _NS)
  base = (s * _NC + c) * per_w
  nch = per_w // _B2
  sg = (sg0, sg1)
  ss = (ss0, ss1)
  sdf = (sdf0, sdf1)
  ssc = (ssc0, ssc1)

  pltpu.sync_copy(zrow_hbm.at[pl.ds(r0, _STRIPE)], acc.at[pl.ds(r0, _STRIPE)])
  plsc.subcore_barrier()

  def issue_src(j, b):
    pltpu.async_copy(src_hbm.at[pl.ds(base + j * _B2, _B2)],
                     idx_s.at[b], ss[b])

  def issue_df(j, k, b):
    pltpu.async_copy(dst_hbm.at[pl.ds(base + j * _B2, _B2)],
                     idx_d.at[k], sdf[b])
    pltpu.async_copy(frac_hbm.at[pl.ds(base + j * _B2, _B2)],
                     frb_v.at[b], sdf[b])

  def issue_gather(j, b):
    pltpu.async_copy(y_hbm.at[idx_s.at[b]], grows_v.at[b], sg[b])

  def step(j, k):
    b = k % 2
    nb = 1 - b
    pltpu.make_async_copy(y_hbm.at[idx_s.at[b]], grows_v.at[b], sg[b]).wait()

    @pl.when(j + 1 < nch)
    def _():
      pltpu.make_async_copy(src_hbm.at[pl.ds(0, _B2)], idx_s.at[nb],
                            ss[nb]).wait()
      issue_gather(j + 1, nb)

    @pl.when(j + 2 < nch)
    def _():
      issue_src(j + 2, b)

    pltpu.make_async_copy(dst_hbm.at[pl.ds(0, _B2)], idx_d.at[k],
                          sdf[b]).wait()
    pltpu.make_async_copy(frac_hbm.at[pl.ds(0, _B2)], frb_v.at[b],
                          sdf[b]).wait()

    @pl.when(j >= 2)
    def _():
      pltpu.make_async_copy(crows_v.at[b], acc.at[idx_d.at[(k + 2) % 4]],
                            ssc[b]).wait()

    @pl.loop(0, _B2)
    def _(e):
      fs = frb_v[b, e]
      for cc in range(_OUT // _L):
        a = grows_v[b, e, pl.ds(cc * _L, _L)]
        bb = grows_v[b, e, pl.ds(_OUT + cc * _L, _L)]
        crows_v[b, e, pl.ds(cc * _L, _L)] = a + fs * bb

    pltpu.async_copy(crows_v.at[b], acc.at[idx_d.at[k]], ssc[b], add=True)

    @pl.when(j + 2 < nch)
    def _():
      issue_df(j + 2, (k + 2) % 4, b)

  issue_src(0, 0)
  issue_src(1, 1)
  issue_df(0, 0, 0)
  issue_df(1, 1, 1)
  pltpu.make_async_copy(src_hbm.at[pl.ds(0, _B2)], idx_s.at[0], ss0).wait()
  issue_gather(0, 0)

  @pl.loop(0, nch // 4)
  def _(jj):
    j0 = 4 * jj
    step(j0, 0)
    step(j0 + 1, 1)
    step(j0 + 2, 2)
    step(j0 + 3, 3)

  step(jnp.int32(nch - 2), 0)
  step(jnp.int32(nch - 1), 1)

  pltpu.make_async_copy(crows_v.at[0], acc.at[idx_d.at[0]], ssc[0]).wait()
  pltpu.make_async_copy(crows_v.at[1], acc.at[idx_d.at[1]], ssc[1]).wait()

  plsc.subcore_barrier()
  pltpu.sync_copy(acc.at[pl.ds(r0, _STRIPE)], p_hbm.at[c, pl.ds(r0, _STRIPE)])


_BN = 1000


def _tc1_body(s0_ref, s1_ref, deg_ref, x_ref, w0_ref, dw_ref, r1_ref, b1_ref,
              w2c_ref, h_ref, y_ref):
  recip = 1.0 / jnp.maximum(deg_ref[...], 1.0)
  agg = (jnp.dot(s0_ref[0], w0_ref[...], preferred_element_type=jnp.float32)
         + jnp.dot(s1_ref[0], dw_ref[...],
                   preferred_element_type=jnp.float32))
  t = (agg * recip
       + jnp.dot(x_ref[...], r1_ref[...], preferred_element_type=jnp.float32)
       + b1_ref[...])
  h = jnp.where(t > 0, t, jnp.exp(t) - 1.0)
  h_ref[...] = h
  y_ref[...] = jnp.dot(h, w2c_ref[...], preferred_element_type=jnp.float32)


def _tc2_body(p0_ref, p1_ref, deg_ref, h_ref, r2_ref, b2_ref, o_ref):
  recip = 1.0 / jnp.maximum(deg_ref[...], 1.0)
  o = ((p0_ref[0] + p1_ref[0]) * recip
       + jnp.dot(h_ref[...], r2_ref[...], preferred_element_type=jnp.float32)
       + b2_ref[...])
  m = jnp.max(o, axis=1, keepdims=True)
  o_ref[...] = (o - m) - jnp.log(
      jnp.sum(jnp.exp(o - m), axis=1, keepdims=True))


def kernel(x, edge_index, edge_attr, W1, root1, b1, W2, root2, b2):
  src = edge_index[0]
  dst = edge_index[1]
  frac = jnp.broadcast_to(edge_attr[:, 0:1], (_E, _L))
  zrow = jnp.zeros((_NP, _IN), jnp.float32)
  zdeg = jnp.zeros((_NP,), jnp.float32)

  sc1 = pl.kernel(
      _sc_layer1_body,
      out_type=[jax.ShapeDtypeStruct((_NC, _NP, _IN), jnp.float32),
                jax.ShapeDtypeStruct((_NP,), jnp.float32)],
      mesh=_mesh,
      scratch_types=[
          pltpu.VMEM((2, _B1), jnp.int32),
          pltpu.VMEM((2, _B1), jnp.int32),
          pltpu.VMEM((2, _B1, _L), jnp.float32),
          pltpu.VMEM((2, _B1, _IN), jnp.float32),
          pltpu.VMEM((_B1,), jnp.float32),
          pltpu.VMEM_SHARED((_NP, _IN), jnp.float32),
          pltpu.VMEM_SHARED((_NP,), jnp.float32),
          pltpu.SemaphoreType.DMA,
          pltpu.SemaphoreType.DMA,
          pltpu.SemaphoreType.DMA,
          pltpu.SemaphoreType.DMA,
          pltpu.SemaphoreType.DMA,
          pltpu.SemaphoreType.DMA,
      ],
  )
  S, deg1 = sc1(x, src, dst, frac, zrow, zdeg)
  degc = deg1[:_N].reshape(_N, 1)

  grid = (_N // _BN,)
  h, Y = pl.pallas_call(
      _tc1_body,
      grid=grid,
      in_specs=[
          pl.BlockSpec((1, _BN, _IN), lambda i: (0, i, 0)),
          pl.BlockSpec((1, _BN, _IN), lambda i: (1, i, 0)),
          pl.BlockSpec((_BN, 1), lambda i: (i, 0)),
          pl.BlockSpec((_BN, _IN), lambda i: (i, 0)),
          pl.BlockSpec((_IN, _HID), lambda i: (0, 0)),
          pl.BlockSpec((_IN, _HID), lambda i: (0, 0)),
          pl.BlockSpec((_IN, _HID), lambda i: (0, 0)),
          pl.BlockSpec((1, _HID), lambda i: (0, 0)),
          pl.BlockSpec((_HID, 2 * _OUT), lambda i: (0, 0)),
      ],
      out_specs=[
          pl.BlockSpec((_BN, _HID), lambda i: (i, 0)),
          pl.BlockSpec((_BN, 2 * _OUT), lambda i: (i, 0)),
      ],
      out_shape=[
          jax.ShapeDtypeStruct((_N, _HID), jnp.float32),
          jax.ShapeDtypeStruct((_N, 2 * _OUT), jnp.float32),
      ],
  )(S, S, degc, x, W1[0], W1[1] - W1[0], root1,
    b1.reshape(1, _HID), jnp.concatenate([W2[0], W2[1] - W2[0]], axis=1))

  sc2 = pl.kernel(
      _sc_layer2_body,
      out_type=jax.ShapeDtypeStruct((_NC, _NP, _OUT), jnp.float32),
      mesh=_mesh,
      scratch_types=[
          pltpu.VMEM((2, _B2), jnp.int32),
          pltpu.VMEM((4, _B2), jnp.int32),
          pltpu.VMEM((2, _B2, _L), jnp.float32),
          pltpu.VMEM((2, _B2, 2 * _OUT), jnp.float32),
          pltpu.VMEM((2, _B2, _OUT), jnp.float32),
          pltpu.VMEM_SHARED((_NP, _OUT), jnp.float32),
          pltpu.SemaphoreType.DMA,
          pltpu.SemaphoreType.DMA,
          pltpu.SemaphoreType.DMA,
          pltpu.SemaphoreType.DMA,
          pltpu.SemaphoreType.DMA,
          pltpu.SemaphoreType.DMA,
          pltpu.SemaphoreType.DMA,
          pltpu.SemaphoreType.DMA,
      ],
  )
  P = sc2(Y, src, dst, frac, zrow)

  out = pl.pallas_call(
      _tc2_body,
      grid=grid,
      in_specs=[
          pl.BlockSpec((1, _BN, _OUT), lambda i: (0, i, 0)),
          pl.BlockSpec((1, _BN, _OUT), lambda i: (1, i, 0)),
          pl.BlockSpec((_BN, 1), lambda i: (i, 0)),
          pl.BlockSpec((_BN, _HID), lambda i: (i, 0)),
          pl.BlockSpec((_HID, _OUT), lambda i: (0, 0)),
          pl.BlockSpec((1, _OUT), lambda i: (0, 0)),
      ],
      out_specs=pl.BlockSpec((_BN, _OUT), lambda i: (i, 0)),
      out_shape=jax.ShapeDtypeStruct((_N, _OUT), jnp.float32),
  )(P, P, degc, h, root2, b2.reshape(1, _OUT))

  return out


# speedup vs baseline: 5.4949x; 1.1982x over previous
import jax
import jax.numpy as jnp
from jax import lax
from jax.experimental import pallas as pl
from jax.experimental.pallas import tpu as pltpu
from jax.experimental.pallas import tpu_sc as plsc

_N = 10000
_E = 320000
_IN = 128
_HID = 256
_OUT = 128
_NC = 2
_NS = 16
_L = 16
_B1 = 80
_B2 = 40
_NP = 10240
_STRIPE = _NP // _NS

_mesh = plsc.VectorSubcoreMesh(core_axis_name="c", subcore_axis_name="s")


def _sc_layer1_body(x_hbm, src_hbm, dst_hbm, frac_hbm, zrow_hbm, zdeg_hbm,
                    s_hbm, deg_hbm,
                    idx_s, idx_d, frb_v, rows_v, ones_v, acc, dacc,
                    sg0, sg1, ss0, ss1, sdf0, sdf1):
  c = lax.axis_index("c")
  s = lax.axis_index("s")
  r0 = s * _STRIPE
  per_tile = _E // _NS
  base = s * per_tile
  nch = per_tile // _B1
  sg = (sg0, sg1)
  ss = (ss0, ss1)
  sdf = (sdf0, sdf1)

  pltpu.sync_copy(zrow_hbm.at[pl.ds(r0, _STRIPE)], acc.at[pl.ds(r0, _STRIPE)])

  @pl.when(c == 0)
  def _():
    pltpu.sync_copy(zdeg_hbm.at[pl.ds(r0, _STRIPE)],
                    dacc.at[pl.ds(r0, _STRIPE)])

  @pl.loop(0, _B1 // _L)
  def _(g):
    ones_v[pl.ds(g * _L, _L)] = jnp.full((_L,), 1.0, jnp.float32)

  plsc.subcore_barrier()

  def issue_src(j, b):
    pltpu.async_copy(src_hbm.at[pl.ds(base + j * _B1, _B1)],
                     idx_s.at[b], ss[b])

  def issue_df(j, b):
    pltpu.async_copy(dst_hbm.at[pl.ds(base + j * _B1, _B1)],
                     idx_d.at[b], sdf[b])

    @pl.when(c == 1)
    def _():
      pltpu.async_copy(frac_hbm.at[pl.ds(base + j * _B1, _B1)],
                       frb_v.at[b], sdf[b])

  def issue_gather(j, b):
    pltpu.async_copy(x_hbm.at[idx_s.at[b]], rows_v.at[b], sg[b])

  def step(j, b, nb):
    pltpu.make_async_copy(x_hbm.at[idx_s.at[b]], rows_v.at[b], sg[b]).wait()

    @pl.when(j + 1 < nch)
    def _():
      pltpu.make_async_copy(src_hbm.at[pl.ds(0, _B1)], idx_s.at[nb],
                            ss[nb]).wait()
      issue_gather(j + 1, nb)

    @pl.when(j + 2 < nch)
    def _():
      issue_src(j + 2, b)

    pltpu.make_async_copy(dst_hbm.at[pl.ds(0, _B1)], idx_d.at[b],
                          sdf[b]).wait()

    @pl.when(c == 1)
    def _():
      pltpu.make_async_copy(frac_hbm.at[pl.ds(0, _B1)], frb_v.at[b],
                            sdf[b]).wait()

      @pl.loop(0, _B1)
      def _(e):
        fs = frb_v[b, e]
        for cc in range(_IN // _L):
          rows_v[b, e, pl.ds(cc * _L, _L)] = (
              rows_v[b, e, pl.ds(cc * _L, _L)] * fs)

    pltpu.sync_copy(rows_v.at[b], acc.at[idx_d.at[b]], add=True)

    @pl.when(c == 0)
    def _():
      pltpu.sync_copy(ones_v, dacc.at[idx_d.at[b]], add=True)

    @pl.when(j + 2 < nch)
    def _():
      issue_df(j + 2, b)

  issue_src(0, 0)
  issue_src(1, 1)
  issue_df(0, 0)
  issue_df(1, 1)
  pltpu.make_async_copy(src_hbm.at[pl.ds(0, _B1)], idx_s.at[0], ss0).wait()
  issue_gather(0, 0)

  @pl.loop(0, nch // 2)
  def _(jj):
    j0 = 2 * jj
    step(j0, 0, 1)
    step(j0 + 1, 1, 0)

  plsc.subcore_barrier()
  pltpu.sync_copy(acc.at[pl.ds(r0, _STRIPE)], s_hbm.at[c, pl.ds(r0, _STRIPE)])

  @pl.when(c == 0)
  def _():
    pltpu.sync_copy(dacc.at[pl.ds(r0, _STRIPE)],
                    deg_hbm.at[pl.ds(r0, _STRIPE)])


def _sc_layer2_body(y0_hbm, yd_hbm, src_hbm, dst_hbm, frac_hbm, zrow_hbm,
                    p_hbm,
                    idx_s, idx_d, frb_v, rows_v, acc,
                    sg0, sg1, ss0, ss1, sdf0, sdf1):
  c = lax.axis_index("c")
  s = lax.axis_index("s")
  r0 = s * _STRIPE
  per_tile = _E // _NS
  base = s * per_tile
  nch = per_tile // _B1
  sg = (sg0, sg1)
  ss = (ss0, ss1)
  sdf = (sdf0, sdf1)

  pltpu.sync_copy(zrow_hbm.at[pl.ds(r0, _STRIPE)], acc.at[pl.ds(r0, _STRIPE)])
  plsc.subcore_barrier()

  def issue_src(j, b):
    pltpu.async_copy(src_hbm.at[pl.ds(base + j * _B1, _B1)],
                     idx_s.at[b], ss[b])

  def issue_df(j, b):
    pltpu.async_copy(dst_hbm.at[pl.ds(base + j * _B1, _B1)],
                     idx_d.at[b], sdf[b])

    @pl.when(c == 1)
    def _():
      pltpu.async_copy(frac_hbm.at[pl.ds(base + j * _B1, _B1)],
                       frb_v.at[b], sdf[b])

  def issue_gather(j, b):
    @pl.when(c == 0)
    def _():
      pltpu.async_copy(y0_hbm.at[idx_s.at[b]], rows_v.at[b], sg[b])

    @pl.when(c == 1)
    def _():
      pltpu.async_copy(yd_hbm.at[idx_s.at[b]], rows_v.at[b], sg[b])

  def step(j, b, nb):
    pltpu.make_async_copy(y0_hbm.at[idx_s.at[b]], rows_v.at[b], sg[b]).wait()

    @pl.when(j + 1 < nch)
    def _():
      pltpu.make_async_copy(src_hbm.at[pl.ds(0, _B1)], idx_s.at[nb],
                            ss[nb]).wait()
      issue_gather(j + 1, nb)

    @pl.when(j + 2 < nch)
    def _():
      issue_src(j + 2, b)

    pltpu.make_async_copy(dst_hbm.at[pl.ds(0, _B1)], idx_d.at[b],
                          sdf[b]).wait()

    @pl.when(c == 1)
    def _():
      pltpu.make_async_copy(frac_hbm.at[pl.ds(0, _B1)], frb_v.at[b],
                            sdf[b]).wait()

      @pl.loop(0, _B1)
      def _(e):
        fs = frb_v[b, e]
        for cc in range(_OUT // _L):
          rows_v[b, e, pl.ds(cc * _L, _L)] = (
              rows_v[b, e, pl.ds(cc * _L, _L)] * fs)

    pltpu.sync_copy(rows_v.at[b], acc.at[idx_d.at[b]], add=True)

    @pl.when(j + 2 < nch)
    def _():
      issue_df(j + 2, b)

  issue_src(0, 0)
  issue_src(1, 1)
  issue_df(0, 0)
  issue_df(1, 1)
  pltpu.make_async_copy(src_hbm.at[pl.ds(0, _B1)], idx_s.at[0], ss0).wait()
  issue_gather(0, 0)

  @pl.loop(0, nch // 2)
  def _(jj):
    j0 = 2 * jj
    step(j0, 0, 1)
    step(j0 + 1, 1, 0)

  plsc.subcore_barrier()
  pltpu.sync_copy(acc.at[pl.ds(r0, _STRIPE)], p_hbm.at[c, pl.ds(r0, _STRIPE)])


_BN = 1000


def _tc1_body(s0_ref, s1_ref, deg_ref, x_ref, w0_ref, dw_ref, r1_ref, b1_ref,
              w20_ref, w2d_ref, h_ref, y0_ref, yd_ref):
  recip = 1.0 / jnp.maximum(deg_ref[...], 1.0)
  agg = (jnp.dot(s0_ref[0], w0_ref[...], preferred_element_type=jnp.float32)
         + jnp.dot(s1_ref[0], dw_ref[...],
                   preferred_element_type=jnp.float32))
  t = (agg * recip
       + jnp.dot(x_ref[...], r1_ref[...], preferred_element_type=jnp.float32)
       + b1_ref[...])
  h = jnp.where(t > 0, t, jnp.exp(t) - 1.0)
  h_ref[...] = h
  y0_ref[...] = jnp.dot(h, w20_ref[...], preferred_element_type=jnp.float32)
  yd_ref[...] = jnp.dot(h, w2d_ref[...], preferred_element_type=jnp.float32)


def _tc2_body(p0_ref, p1_ref, deg_ref, h_ref, r2_ref, b2_ref, o_ref):
  recip = 1.0 / jnp.maximum(deg_ref[...], 1.0)
  o = ((p0_ref[0] + p1_ref[0]) * recip
       + jnp.dot(h_ref[...], r2_ref[...], preferred_element_type=jnp.float32)
       + b2_ref[...])
  m = jnp.max(o, axis=1, keepdims=True)
  o_ref[...] = (o - m) - jnp.log(
      jnp.sum(jnp.exp(o - m), axis=1, keepdims=True))


def kernel(x, edge_index, edge_attr, W1, root1, b1, W2, root2, b2):
  src = edge_index[0]
  dst = edge_index[1]
  frac = jnp.broadcast_to(edge_attr[:, 0:1], (_E, _L))
  zrow = jnp.zeros((_NP, _IN), jnp.float32)
  zdeg = jnp.zeros((_NP,), jnp.float32)

  sc1 = pl.kernel(
      _sc_layer1_body,
      out_type=[jax.ShapeDtypeStruct((_NC, _NP, _IN), jnp.float32),
                jax.ShapeDtypeStruct((_NP,), jnp.float32)],
      mesh=_mesh,
      scratch_types=[
          pltpu.VMEM((2, _B1), jnp.int32),
          pltpu.VMEM((2, _B1), jnp.int32),
          pltpu.VMEM((2, _B1, _L), jnp.float32),
          pltpu.VMEM((2, _B1, _IN), jnp.float32),
          pltpu.VMEM((_B1,), jnp.float32),
          pltpu.VMEM_SHARED((_NP, _IN), jnp.float32),
          pltpu.VMEM_SHARED((_NP,), jnp.float32),
          pltpu.SemaphoreType.DMA,
          pltpu.SemaphoreType.DMA,
          pltpu.SemaphoreType.DMA,
          pltpu.SemaphoreType.DMA,
          pltpu.SemaphoreType.DMA,
          pltpu.SemaphoreType.DMA,
      ],
  )
  S, deg1 = sc1(x, src, dst, frac, zrow, zdeg)
  degc = deg1[:_N].reshape(_N, 1)

  grid = (_N // _BN,)
  h, Y0, Yd = pl.pallas_call(
      _tc1_body,
      grid=grid,
      in_specs=[
          pl.BlockSpec((1, _BN, _IN), lambda i: (0, i, 0)),
          pl.BlockSpec((1, _BN, _IN), lambda i: (1, i, 0)),
          pl.BlockSpec((_BN, 1), lambda i: (i, 0)),
          pl.BlockSpec((_BN, _IN), lambda i: (i, 0)),
          pl.BlockSpec((_IN, _HID), lambda i: (0, 0)),
          pl.BlockSpec((_IN, _HID), lambda i: (0, 0)),
          pl.BlockSpec((_IN, _HID), lambda i: (0, 0)),
          pl.BlockSpec((1, _HID), lambda i: (0, 0)),
          pl.BlockSpec((_HID, _OUT), lambda i: (0, 0)),
          pl.BlockSpec((_HID, _OUT), lambda i: (0, 0)),
      ],
      out_specs=[
          pl.BlockSpec((_BN, _HID), lambda i: (i, 0)),
          pl.BlockSpec((_BN, _OUT), lambda i: (i, 0)),
          pl.BlockSpec((_BN, _OUT), lambda i: (i, 0)),
      ],
      out_shape=[
          jax.ShapeDtypeStruct((_N, _HID), jnp.float32),
          jax.ShapeDtypeStruct((_N, _OUT), jnp.float32),
          jax.ShapeDtypeStruct((_N, _OUT), jnp.float32),
      ],
  )(S, S, degc, x, W1[0], W1[1] - W1[0], root1,
    b1.reshape(1, _HID), W2[0], W2[1] - W2[0])

  sc2 = pl.kernel(
      _sc_layer2_body,
      out_type=jax.ShapeDtypeStruct((_NC, _NP, _OUT), jnp.float32),
      mesh=_mesh,
      scratch_types=[
          pltpu.VMEM((2, _B1), jnp.int32),
          pltpu.VMEM((2, _B1), jnp.int32),
          pltpu.VMEM((2, _B1, _L), jnp.float32),
          pltpu.VMEM((2, _B1, _OUT), jnp.float32),
          pltpu.VMEM_SHARED((_NP, _OUT), jnp.float32),
          pltpu.SemaphoreType.DMA,
          pltpu.SemaphoreType.DMA,
          pltpu.SemaphoreType.DMA,
          pltpu.SemaphoreType.DMA,
          pltpu.SemaphoreType.DMA,
          pltpu.SemaphoreType.DMA,
      ],
  )
  P = sc2(Y0, Yd, src, dst, frac, zrow)

  out = pl.pallas_call(
      _tc2_body,
      grid=grid,
      in_specs=[
          pl.BlockSpec((1, _BN, _OUT), lambda i: (0, i, 0)),
          pl.BlockSpec((1, _BN, _OUT), lambda i: (1, i, 0)),
          pl.BlockSpec((_BN, 1), lambda i: (i, 0)),
          pl.BlockSpec((_BN, _HID), lambda i: (i, 0)),
          pl.BlockSpec((_HID, _OUT), lambda i: (0, 0)),
          pl.BlockSpec((1, _OUT), lambda i: (0, 0)),
      ],
      out_specs=pl.BlockSpec((_BN, _OUT), lambda i: (i, 0)),
      out_shape=jax.ShapeDtypeStruct((_N, _OUT), jnp.float32),
  )(P, P, degc, h, root2, b2.reshape(1, _OUT))

  return out


# speedup vs baseline: 5.7177x; 1.0406x over previous
import jax
import jax.numpy as jnp
from jax import lax
from jax.experimental import pallas as pl
from jax.experimental.pallas import tpu as pltpu
from jax.experimental.pallas import tpu_sc as plsc

_N = 10000
_E = 320000
_IN = 128
_HID = 256
_OUT = 128
_NC = 2
_NS = 16
_L = 16
_B1 = 80
_B2 = 40
_NP = 10240
_STRIPE = _NP // _NS

_mesh = plsc.VectorSubcoreMesh(core_axis_name="c", subcore_axis_name="s")


def _make_sc_body(with_deg):
  def body(t0_hbm, t1_hbm, src_hbm, dst_hbm, frac_hbm, zrow_hbm, zdeg_hbm,
           out_hbm, deg_hbm,
           idx_s, idx_d, frb_v, rows_v, ones_v, acc, dacc,
           sg0, sg1, ss0, ss1, sdf0, sdf1, ssc0, ssc1):
    c = lax.axis_index("c")
    s = lax.axis_index("s")
    r0 = s * _STRIPE
    per_tile = _E // _NS
    base = s * per_tile
    nch = per_tile // _B1
    sg = (sg0, sg1)
    ss = (ss0, ss1)
    sdf = (sdf0, sdf1)
    ssc = (ssc0, ssc1)

    pltpu.sync_copy(zrow_hbm.at[pl.ds(r0, _STRIPE)],
                    acc.at[pl.ds(r0, _STRIPE)])
    if with_deg:
      @pl.when(c == 0)
      def _():
        pltpu.sync_copy(zdeg_hbm.at[pl.ds(r0, _STRIPE)],
                        dacc.at[pl.ds(r0, _STRIPE)])

      @pl.loop(0, _B1 // _L)
      def _(g):
        ones_v[pl.ds(g * _L, _L)] = jnp.full((_L,), 1.0, jnp.float32)

    plsc.subcore_barrier()

    def issue_src(j, b):
      pltpu.async_copy(src_hbm.at[pl.ds(base + j * _B1, _B1)],
                       idx_s.at[b], ss[b])

    def issue_df(j, k, b):
      pltpu.async_copy(dst_hbm.at[pl.ds(base + j * _B1, _B1)],
                       idx_d.at[k], sdf[b])

      @pl.when(c == 1)
      def _():
        pltpu.async_copy(frac_hbm.at[pl.ds(base + j * _B1, _B1)],
                         frb_v.at[b], sdf[b])

    def issue_gather(j, b):
      @pl.when(c == 0)
      def _():
        pltpu.async_copy(t0_hbm.at[idx_s.at[b]], rows_v.at[b], sg[b])

      @pl.when(c == 1)
      def _():
        pltpu.async_copy(t1_hbm.at[idx_s.at[b]], rows_v.at[b], sg[b])

    def drain_scatter(k, b):
      pltpu.make_async_copy(rows_v.at[b], acc.at[idx_d.at[k]], ssc[b]).wait()
      if with_deg:
        @pl.when(c == 0)
        def _():
          pltpu.make_async_copy(ones_v, dacc.at[idx_d.at[k]], ssc[b]).wait()

    def step(j, k):
      b = k % 2
      nb = 1 - b
      pltpu.make_async_copy(t0_hbm.at[idx_s.at[b]], rows_v.at[b],
                            sg[b]).wait()

      @pl.when(j + 1 < nch)
      def _():
        @pl.when(j >= 1)
        def _():
          drain_scatter((k + 3) % 4, nb)
        pltpu.make_async_copy(src_hbm.at[pl.ds(0, _B1)], idx_s.at[nb],
                              ss[nb]).wait()
        issue_gather(j + 1, nb)

      @pl.when(j + 2 < nch)
      def _():
        issue_src(j + 2, b)

      pltpu.make_async_copy(dst_hbm.at[pl.ds(0, _B1)], idx_d.at[k],
                            sdf[b]).wait()

      @pl.when(c == 1)
      def _():
        pltpu.make_async_copy(frac_hbm.at[pl.ds(0, _B1)], frb_v.at[b],
                              sdf[b]).wait()

        @pl.loop(0, _B1)
        def _(e):
          fs = frb_v[b, e]
          for cc in range(_IN // _L):
            rows_v[b, e, pl.ds(cc * _L, _L)] = (
                rows_v[b, e, pl.ds(cc * _L, _L)] * fs)

      pltpu.async_copy(rows_v.at[b], acc.at[idx_d.at[k]], ssc[b], add=True)
      if with_deg:
        @pl.when(c == 0)
        def _():
          pltpu.async_copy(ones_v, dacc.at[idx_d.at[k]], ssc[b], add=True)

      @pl.when(j + 2 < nch)
      def _():
        issue_df(j + 2, (k + 2) % 4, b)

    issue_src(0, 0)
    issue_src(1, 1)
    issue_df(0, 0, 0)
    issue_df(1, 1, 1)
    pltpu.make_async_copy(src_hbm.at[pl.ds(0, _B1)], idx_s.at[0], ss0).wait()
    issue_gather(0, 0)

    @pl.loop(0, 250 // 4)
    def _(jj):
      j0 = 4 * jj
      step(j0, 0)
      step(j0 + 1, 1)
      step(j0 + 2, 2)
      step(j0 + 3, 3)

    step(jnp.int32(248), 0)
    step(jnp.int32(249), 1)
    drain_scatter(0, 0)
    drain_scatter(1, 1)

    plsc.subcore_barrier()
    pltpu.sync_copy(acc.at[pl.ds(r0, _STRIPE)],
                    out_hbm.at[c, pl.ds(r0, _STRIPE)])
    if with_deg:
      @pl.when(c == 0)
      def _():
        pltpu.sync_copy(dacc.at[pl.ds(r0, _STRIPE)],
                        deg_hbm.at[pl.ds(r0, _STRIPE)])

  return body


def _sc_layer1_body(x_hbm, src_hbm, dst_hbm, frac_hbm, zrow_hbm, zdeg_hbm,
                    s_hbm, deg_hbm, *scratch):
  _make_sc_body(True)(x_hbm, x_hbm, src_hbm, dst_hbm, frac_hbm, zrow_hbm,
                      zdeg_hbm, s_hbm, deg_hbm, *scratch)


def _sc_layer2_body(y0_hbm, yd_hbm, src_hbm, dst_hbm, frac_hbm, zrow_hbm,
                    p_hbm, *scratch):
  args = list(scratch)
  _make_sc_body(False)(y0_hbm, yd_hbm, src_hbm, dst_hbm, frac_hbm, zrow_hbm,
                       zrow_hbm, p_hbm, p_hbm, *args)


_BN = 1000


def _tc1_body(s0_ref, s1_ref, deg_ref, x_ref, w0_ref, dw_ref, r1_ref, b1_ref,
              w20_ref, w2d_ref, h_ref, y0_ref, yd_ref):
  recip = 1.0 / jnp.maximum(deg_ref[...], 1.0)
  agg = (jnp.dot(s0_ref[0], w0_ref[...], preferred_element_type=jnp.float32)
         + jnp.dot(s1_ref[0], dw_ref[...],
                   preferred_element_type=jnp.float32))
  t = (agg * recip
       + jnp.dot(x_ref[...], r1_ref[...], preferred_element_type=jnp.float32)
       + b1_ref[...])
  h = jnp.where(t > 0, t, jnp.exp(t) - 1.0)
  h_ref[...] = h
  y0_ref[...] = jnp.dot(h, w20_ref[...], preferred_element_type=jnp.float32)
  yd_ref[...] = jnp.dot(h, w2d_ref[...], preferred_element_type=jnp.float32)


def _tc2_body(p0_ref, p1_ref, deg_ref, h_ref, r2_ref, b2_ref, o_ref):
  recip = 1.0 / jnp.maximum(deg_ref[...], 1.0)
  o = ((p0_ref[0] + p1_ref[0]) * recip
       + jnp.dot(h_ref[...], r2_ref[...], preferred_element_type=jnp.float32)
       + b2_ref[...])
  m = jnp.max(o, axis=1, keepdims=True)
  o_ref[...] = (o - m) - jnp.log(
      jnp.sum(jnp.exp(o - m), axis=1, keepdims=True))


def kernel(x, edge_index, edge_attr, W1, root1, b1, W2, root2, b2):
  src = edge_index[0]
  dst = edge_index[1]
  frac = jnp.broadcast_to(edge_attr[:, 0:1], (_E, _L))
  zrow = jnp.zeros((_NP, _IN), jnp.float32)
  zdeg = jnp.zeros((_NP,), jnp.float32)

  sc1 = pl.kernel(
      _sc_layer1_body,
      out_type=[jax.ShapeDtypeStruct((_NC, _NP, _IN), jnp.float32),
                jax.ShapeDtypeStruct((_NP,), jnp.float32)],
      mesh=_mesh,
      scratch_types=[
          pltpu.VMEM((2, _B1), jnp.int32),
          pltpu.VMEM((4, _B1), jnp.int32),
          pltpu.VMEM((2, _B1, _L), jnp.float32),
          pltpu.VMEM((2, _B1, _IN), jnp.float32),
          pltpu.VMEM((_B1,), jnp.float32),
          pltpu.VMEM_SHARED((_NP, _IN), jnp.float32),
          pltpu.VMEM_SHARED((_NP,), jnp.float32),
          pltpu.SemaphoreType.DMA,
          pltpu.SemaphoreType.DMA,
          pltpu.SemaphoreType.DMA,
          pltpu.SemaphoreType.DMA,
          pltpu.SemaphoreType.DMA,
          pltpu.SemaphoreType.DMA,
          pltpu.SemaphoreType.DMA,
          pltpu.SemaphoreType.DMA,
      ],
  )
  S, deg1 = sc1(x, src, dst, frac, zrow, zdeg)
  degc = deg1[:_N].reshape(_N, 1)

  grid = (_N // _BN,)
  h, Y0, Yd = pl.pallas_call(
      _tc1_body,
      grid=grid,
      in_specs=[
          pl.BlockSpec((1, _BN, _IN), lambda i: (0, i, 0)),
          pl.BlockSpec((1, _BN, _IN), lambda i: (1, i, 0)),
          pl.BlockSpec((_BN, 1), lambda i: (i, 0)),
          pl.BlockSpec((_BN, _IN), lambda i: (i, 0)),
          pl.BlockSpec((_IN, _HID), lambda i: (0, 0)),
          pl.BlockSpec((_IN, _HID), lambda i: (0, 0)),
          pl.BlockSpec((_IN, _HID), lambda i: (0, 0)),
          pl.BlockSpec((1, _HID), lambda i: (0, 0)),
          pl.BlockSpec((_HID, _OUT), lambda i: (0, 0)),
          pl.BlockSpec((_HID, _OUT), lambda i: (0, 0)),
      ],
      out_specs=[
          pl.BlockSpec((_BN, _HID), lambda i: (i, 0)),
          pl.BlockSpec((_BN, _OUT), lambda i: (i, 0)),
          pl.BlockSpec((_BN, _OUT), lambda i: (i, 0)),
      ],
      out_shape=[
          jax.ShapeDtypeStruct((_N, _HID), jnp.float32),
          jax.ShapeDtypeStruct((_N, _OUT), jnp.float32),
          jax.ShapeDtypeStruct((_N, _OUT), jnp.float32),
      ],
  )(S, S, degc, x, W1[0], W1[1] - W1[0], root1,
    b1.reshape(1, _HID), W2[0], W2[1] - W2[0])

  sc2 = pl.kernel(
      _sc_layer2_body,
      out_type=jax.ShapeDtypeStruct((_NC, _NP, _OUT), jnp.float32),
      mesh=_mesh,
      scratch_types=[
          pltpu.VMEM((2, _B1), jnp.int32),
          pltpu.VMEM((4, _B1), jnp.int32),
          pltpu.VMEM((2, _B1, _L), jnp.float32),
          pltpu.VMEM((2, _B1, _OUT), jnp.float32),
          pltpu.VMEM((_B1,), jnp.float32),
          pltpu.VMEM_SHARED((_NP, _OUT), jnp.float32),
          pltpu.VMEM_SHARED((_NP,), jnp.float32),
          pltpu.SemaphoreType.DMA,
          pltpu.SemaphoreType.DMA,
          pltpu.SemaphoreType.DMA,
          pltpu.SemaphoreType.DMA,
          pltpu.SemaphoreType.DMA,
          pltpu.SemaphoreType.DMA,
          pltpu.SemaphoreType.DMA,
          pltpu.SemaphoreType.DMA,
      ],
  )
  P = sc2(Y0, Yd, src, dst, frac, zrow)

  out = pl.pallas_call(
      _tc2_body,
      grid=grid,
      in_specs=[
          pl.BlockSpec((1, _BN, _OUT), lambda i: (0, i, 0)),
          pl.BlockSpec((1, _BN, _OUT), lambda i: (1, i, 0)),
          pl.BlockSpec((_BN, 1), lambda i: (i, 0)),
          pl.BlockSpec((_BN, _HID), lambda i: (i, 0)),
          pl.BlockSpec((_HID, _OUT), lambda i: (0, 0)),
          pl.BlockSpec((1, _OUT), lambda i: (0, 0)),
      ],
      out_specs=pl.BlockSpec((_BN, _OUT), lambda i: (i, 0)),
      out_shape=jax.ShapeDtypeStruct((_N, _OUT), jnp.float32),
  )(P, P, degc, h, root2, b2.reshape(1, _OUT))

  return out
